# baseline (speedup 1.0000x reference)
import functools

import jax
import jax.numpy as jnp
from jax import lax
from jax.experimental import pallas as pl
from jax.experimental.pallas import tpu as pltpu
from jax.experimental.pallas import tpu_sc as plsc

E = 64
D = 768
F = 1024
T = 2048
C = max(int(round(1.0 * T / E)), 4)
SLOTS = E * C

TB = 256
SB = 256

NC = 2
NS = 16
NW = NC * NS
SPT = SLOTS // NW


def _routing_body(x_ref, wr_ref, slot_ref, gate_ref, carry_ref):
    i = pl.program_id(0)

    @pl.when(i == 0)
    def _():
        carry_ref[...] = jnp.zeros_like(carry_ref)

    x = x_ref[...]
    logits = jnp.dot(x, wr_ref[...], preferred_element_type=jnp.float32)
    m = jnp.max(logits, axis=1, keepdims=True)
    s = jnp.sum(jnp.exp(logits - m), axis=1, keepdims=True)
    gate = 1.0 / s

    iota_e = lax.broadcasted_iota(jnp.int32, (TB, E), 1)
    is_max = logits == m
    e_idx = jnp.min(jnp.where(is_max, iota_e, E), axis=1, keepdims=True)
    oh = (iota_e == e_idx).astype(jnp.float32)

    r = lax.broadcasted_iota(jnp.int32, (TB, TB), 0)
    c = lax.broadcasted_iota(jnp.int32, (TB, TB), 1)
    tri = (c <= r).astype(jnp.float32)
    prefix = jnp.dot(tri, oh, preferred_element_type=jnp.float32) + carry_ref[...]
    carry_ref[...] = carry_ref[...] + jnp.sum(oh, axis=0, keepdims=True)

    pos = jnp.sum(prefix * oh, axis=1, keepdims=True) - 1.0
    valid = pos < C
    posi = jnp.minimum(pos, C - 1).astype(jnp.int32)
    slot = jnp.where(valid, e_idx * C + posi, -1)
    slot_ref[...] = slot.reshape(1, TB)
    gate_ref[...] = gate.reshape(1, TB)


def _routing(x, w_router):
    return pl.pallas_call(
        _routing_body,
        grid=(T // TB,),
        in_specs=[
            pl.BlockSpec((TB, D), lambda i: (i, 0)),
            pl.BlockSpec((D, E), lambda i: (0, 0)),
        ],
        out_specs=[
            pl.BlockSpec((1, TB), lambda i: (0, i)),
            pl.BlockSpec((1, TB), lambda i: (0, i)),
        ],
        out_shape=[
            jax.ShapeDtypeStruct((1, T), jnp.int32),
            jax.ShapeDtypeStruct((1, T), jnp.float32),
        ],
        scratch_shapes=[pltpu.VMEM((1, E), jnp.float32)],
    )(x, w_router)


def _invert_body(slot_ref, tos_ref):
    k = pl.program_id(0)
    slot_d = slot_ref[...]
    s_iota = k * SB + lax.broadcasted_iota(jnp.int32, (SB, T), 0)
    mk = (s_iota == slot_d).astype(jnp.float32)
    t_iota = lax.broadcasted_iota(jnp.int32, (SB, T), 1).astype(jnp.float32)
    tos = jnp.sum(mk * t_iota, axis=1, keepdims=True).astype(jnp.int32)
    cnt = jnp.sum(mk, axis=1, keepdims=True)
    s_col = k * SB + lax.broadcasted_iota(jnp.int32, (SB, 1), 0)
    tos_ref[...] = jnp.where(cnt > 0.0, tos, s_col).reshape(1, SB)


def _invert(slot_row):
    return pl.pallas_call(
        _invert_body,
        grid=(SLOTS // SB,),
        in_specs=[pl.BlockSpec((1, T), lambda k: (0, 0))],
        out_specs=pl.BlockSpec((1, SB), lambda k: (0, k)),
        out_shape=jax.ShapeDtypeStruct((1, SLOTS), jnp.int32),
    )(slot_row)


def _dispatch_body(tos_hbm, x_hbm, ei_hbm, idx_v, rows_v, sem):
    cid = lax.axis_index("c")
    sid = lax.axis_index("s")
    base = (sid * NC + cid) * SPT
    pltpu.sync_copy(tos_hbm.at[pl.ds(base, SPT)], idx_v)
    pltpu.async_copy(x_hbm.at[idx_v], rows_v, sem).wait()
    pltpu.sync_copy(rows_v, ei_hbm.at[pl.ds(base, SPT)])


def _dispatch(tos, x):
    mesh = plsc.VectorSubcoreMesh(
        core_axis_name="c", subcore_axis_name="s", num_cores=NC, num_subcores=NS)
    return pl.kernel(
        _dispatch_body,
        out_type=jax.ShapeDtypeStruct((SLOTS, D), jnp.float32),
        mesh=mesh,
        compiler_params=pltpu.CompilerParams(needs_layout_passes=False),
        scratch_types=[
            pltpu.VMEM((SPT,), jnp.int32),
            pltpu.VMEM((SPT, D), jnp.float32),
            pltpu.SemaphoreType.DMA,
        ],
    )(tos, x)


def _ffn_body(ei_ref, w1_ref, b1_ref, w2_ref, b2_ref, slot_ref, gate_ref,
              y_ref):
    e = pl.program_id(0)
    a = ei_ref[0]
    h = jnp.dot(a, w1_ref[0], preferred_element_type=jnp.float32) + b1_ref[0]
    h = jnp.maximum(h, 0.0)
    o = jnp.dot(h, w2_ref[0], preferred_element_type=jnp.float32) + b2_ref[0]
    c_iota = lax.broadcasted_iota(jnp.int32, (C, T), 0) + e * C
    cet = (c_iota == slot_ref[...]).astype(jnp.float32) * gate_ref[...]
    contrib = lax.dot_general(cet, o, (((0,), (0,)), ((), ())),
                              preferred_element_type=jnp.float32)

    @pl.when(e == 0)
    def _():
        y_ref[...] = contrib

    @pl.when(e > 0)
    def _():
        y_ref[...] = y_ref[...] + contrib


def _ffn(ei, w1, b1, w2, b2, slot_row, gate_row):
    em3 = lambda e: (e, 0, 0)
    return pl.pallas_call(
        _ffn_body,
        grid=(E,),
        in_specs=[
            pl.BlockSpec((1, C, D), em3),
            pl.BlockSpec((1, D, F), em3),
            pl.BlockSpec((1, 1, F), em3),
            pl.BlockSpec((1, F, D), em3),
            pl.BlockSpec((1, 1, D), em3),
            pl.BlockSpec((1, T), lambda e: (0, 0)),
            pl.BlockSpec((1, T), lambda e: (0, 0)),
        ],
        out_specs=pl.BlockSpec((T, D), lambda e: (0, 0)),
        out_shape=jax.ShapeDtypeStruct((T, D), jnp.float32),
    )(ei, w1, b1, w2, b2, slot_row, gate_row)


def kernel(inputs, W_router, W1, b1, W2, b2):
    Bv, Sv, d = inputs.shape
    x = inputs.reshape(T, D)
    slot_row, gate_row = _routing(x, W_router)
    tos = _invert(slot_row)
    ei = _dispatch(tos.reshape(SLOTS), x)
    y = _ffn(ei.reshape(E, C, D), W1, b1.reshape(E, 1, F),
             W2, b2.reshape(E, 1, D), slot_row, gate_row)
    return y.reshape(Bv, Sv, d)


# speedup vs baseline: 1.2110x; 1.2110x over previous
import functools

import jax
import jax.numpy as jnp
from jax import lax
from jax.experimental import pallas as pl
from jax.experimental.pallas import tpu as pltpu
from jax.experimental.pallas import tpu_sc as plsc

E = 64
D = 768
F = 1024
T = 2048
C = max(int(round(1.0 * T / E)), 4)
SLOTS = E * C

TB = 256

NC = 2
NS = 16
L = 16
NW = NC * NS
SPT = SLOTS // NW
TPW = T // NW


SB = 256
PH = T // TB


def _route_invert_body(x_ref, wr_ref, slot_ref, gate_ref, tos_ref, gs_ref,
                       carry_ref, slot_sc, gate_sc):
    i = pl.program_id(0)

    @pl.when(i == 0)
    def _():
        carry_ref[...] = jnp.zeros_like(carry_ref)

    @pl.when(i < PH)
    def _():
        x = x_ref[...]
        logits = jnp.dot(x, wr_ref[...], preferred_element_type=jnp.float32)
        m = jnp.max(logits, axis=1, keepdims=True)
        s = jnp.sum(jnp.exp(logits - m), axis=1, keepdims=True)
        gate = 1.0 / s

        iota_e = lax.broadcasted_iota(jnp.int32, (TB, E), 1)
        is_max = logits == m
        e_idx = jnp.min(jnp.where(is_max, iota_e, E), axis=1, keepdims=True)
        oh = (iota_e == e_idx).astype(jnp.float32)

        r = lax.broadcasted_iota(jnp.int32, (TB, TB), 0)
        c = lax.broadcasted_iota(jnp.int32, (TB, TB), 1)
        tri = (c <= r).astype(jnp.float32)
        prefix = jnp.dot(tri, oh, preferred_element_type=jnp.float32) + carry_ref[...]
        carry_ref[...] = carry_ref[...] + jnp.sum(oh, axis=0, keepdims=True)

        pos = jnp.sum(prefix * oh, axis=1, keepdims=True) - 1.0
        valid = pos < C
        posi = jnp.minimum(pos, C - 1).astype(jnp.int32)
        r_iota = lax.broadcasted_iota(jnp.int32, (TB, 1), 0)
        slot = jnp.where(valid, e_idx * C + posi + C, r_iota & (C - 1))
        srow = slot.reshape(1, TB)
        grow = gate.reshape(1, TB)
        slot_ref[...] = srow
        gate_ref[...] = grow
        idx = i * TB
        slot_sc[0, pl.ds(idx, TB)] = srow[0, :]
        gate_sc[0, pl.ds(idx, TB)] = grow[0, :]

    @pl.when(i >= PH)
    def _():
        k = i - PH
        slot_d = slot_sc[...] - C
        s_iota = k * SB + lax.broadcasted_iota(jnp.int32, (SB, T), 0)
        mk = (s_iota == slot_d).astype(jnp.float32)
        t_iota = lax.broadcasted_iota(jnp.int32, (SB, T), 1).astype(jnp.float32)
        tos = jnp.sum(mk * t_iota, axis=1, keepdims=True).astype(jnp.int32)
        cnt = jnp.sum(mk, axis=1, keepdims=True)
        s_col = k * SB + lax.broadcasted_iota(jnp.int32, (SB, 1), 0)
        tos_ref[...] = jnp.where(cnt > 0.0, tos, s_col).reshape(1, SB)
        gs_ref[...] = jnp.sum(mk * gate_sc[...], axis=1, keepdims=True)


def _route_invert(x, w_router):
    return pl.pallas_call(
        _route_invert_body,
        grid=(PH + SLOTS // SB,),
        in_specs=[
            pl.BlockSpec((TB, D), lambda i: (jnp.minimum(i, PH - 1), 0)),
            pl.BlockSpec((D, E), lambda i: (0, 0)),
        ],
        out_specs=[
            pl.BlockSpec((1, TB), lambda i: (0, jnp.minimum(i, PH - 1))),
            pl.BlockSpec((1, TB), lambda i: (0, jnp.minimum(i, PH - 1))),
            pl.BlockSpec((1, SB), lambda i: (0, jnp.maximum(i - PH, 0))),
            pl.BlockSpec((SB, 1), lambda i: (jnp.maximum(i - PH, 0), 0)),
        ],
        out_shape=[
            jax.ShapeDtypeStruct((1, T), jnp.int32),
            jax.ShapeDtypeStruct((1, T), jnp.float32),
            jax.ShapeDtypeStruct((1, SLOTS), jnp.int32),
            jax.ShapeDtypeStruct((SLOTS, 1), jnp.float32),
        ],
        scratch_shapes=[
            pltpu.VMEM((1, E), jnp.float32),
            pltpu.VMEM((1, T), jnp.int32),
            pltpu.VMEM((1, T), jnp.float32),
        ],
    )(x, w_router)


def _dispatch_body(tos_hbm, x_hbm, ei_hbm, idx_v, rows_v, sem):
    cid = lax.axis_index("c")
    sid = lax.axis_index("s")
    base = (sid * NC + cid) * SPT
    pltpu.sync_copy(tos_hbm.at[pl.ds(base, SPT)], idx_v)
    pltpu.async_copy(x_hbm.at[idx_v], rows_v, sem).wait()
    pltpu.sync_copy(rows_v, ei_hbm.at[pl.ds(base, SPT)])


def _dispatch(tos, x):
    mesh = plsc.VectorSubcoreMesh(
        core_axis_name="c", subcore_axis_name="s", num_cores=NC, num_subcores=NS)
    return pl.kernel(
        _dispatch_body,
        out_type=jax.ShapeDtypeStruct((SLOTS, D), jnp.float32),
        mesh=mesh,
        compiler_params=pltpu.CompilerParams(needs_layout_passes=False),
        scratch_types=[
            pltpu.VMEM((SPT,), jnp.int32),
            pltpu.VMEM((SPT, D), jnp.float32),
            pltpu.SemaphoreType.DMA,
        ],
    )(tos, x)


def _ffn_body(ei_ref, w1_ref, b1_ref, w2_ref, b2_ref, gs_ref, eo_ref):
    s = pl.program_id(0)

    @pl.when(s == 0)
    def _():
        eo_ref[...] = jnp.zeros_like(eo_ref)

    @pl.when(s > 0)
    def _():
        a = ei_ref[0]
        h = jnp.dot(a, w1_ref[0], preferred_element_type=jnp.float32) + b1_ref[0]
        h = jnp.maximum(h, 0.0)
        o = jnp.dot(h, w2_ref[0], preferred_element_type=jnp.float32) + b2_ref[0]
        eo_ref[0] = o * gs_ref[0]


def _ffn(ei, w1, b1, w2, b2, gs):
    em = lambda s: (jnp.maximum(s - 1, 0), 0, 0)
    return pl.pallas_call(
        _ffn_body,
        grid=(E + 1,),
        in_specs=[
            pl.BlockSpec((1, C, D), em),
            pl.BlockSpec((1, D, F), em),
            pl.BlockSpec((1, 1, F), em),
            pl.BlockSpec((1, F, D), em),
            pl.BlockSpec((1, 1, D), em),
            pl.BlockSpec((1, C, 1), em),
        ],
        out_specs=pl.BlockSpec((1, C, D), lambda s: (s, 0, 0)),
        out_shape=jax.ShapeDtypeStruct((E + 1, C, D), jnp.float32),
    )(ei, w1, b1, w2, b2, gs)


def _combine_body(slot_hbm, eo_hbm, y_hbm, idx_v, rows_v, sem):
    cid = lax.axis_index("c")
    sid = lax.axis_index("s")
    base = (sid * NC + cid) * TPW
    pltpu.sync_copy(slot_hbm.at[pl.ds(base, TPW)], idx_v)
    pltpu.async_copy(eo_hbm.at[idx_v], rows_v, sem).wait()
    pltpu.sync_copy(rows_v, y_hbm.at[pl.ds(base, TPW)])


def _combine(slot, eo):
    mesh = plsc.VectorSubcoreMesh(
        core_axis_name="c", subcore_axis_name="s", num_cores=NC, num_subcores=NS)
    return pl.kernel(
        _combine_body,
        out_type=jax.ShapeDtypeStruct((T, D), jnp.float32),
        mesh=mesh,
        compiler_params=pltpu.CompilerParams(needs_layout_passes=False),
        scratch_types=[
            pltpu.VMEM((TPW,), jnp.int32),
            pltpu.VMEM((TPW, D), jnp.float32),
            pltpu.SemaphoreType.DMA,
        ],
    )(slot, eo)


def kernel(inputs, W_router, W1, b1, W2, b2):
    Bv, Sv, d = inputs.shape
    x = inputs.reshape(T, D)
    slot2, gate2, tos, gs = _route_invert(x, W_router)
    slot = slot2.reshape(T)
    ei = _dispatch(tos.reshape(SLOTS), x)
    eo = _ffn(ei.reshape(E, C, D), W1, b1.reshape(E, 1, F),
              W2, b2.reshape(E, 1, D), gs.reshape(E, C, 1))
    y = _combine(slot, eo.reshape((E + 1) * C, D))
    return y.reshape(Bv, Sv, d)


# speedup vs baseline: 1.2129x; 1.0016x over previous
import functools

import jax
import jax.numpy as jnp
from jax import lax
from jax.experimental import pallas as pl
from jax.experimental.pallas import tpu as pltpu
from jax.experimental.pallas import tpu_sc as plsc

E = 64
D = 768
F = 1024
T = 2048
C = max(int(round(1.0 * T / E)), 4)
SLOTS = E * C

TB = 256

NC = 2
NS = 16
L = 16
NW = NC * NS
SPT = SLOTS // NW
TPW = T // NW


SB = 256
PH = T // TB


def _route_invert_body(x_ref, wr_ref, slot_ref, gate_ref, tos_ref, gs_ref,
                       carry_ref, slot_sc, gate_sc):
    i = pl.program_id(0)

    @pl.when(i == 0)
    def _():
        carry_ref[...] = jnp.zeros_like(carry_ref)

    @pl.when(i < PH)
    def _():
        x = x_ref[...]
        logits = jnp.dot(x, wr_ref[...], preferred_element_type=jnp.float32)
        m = jnp.max(logits, axis=1, keepdims=True)
        s = jnp.sum(jnp.exp(logits - m), axis=1, keepdims=True)
        gate = 1.0 / s

        iota_e = lax.broadcasted_iota(jnp.int32, (TB, E), 1)
        is_max = logits == m
        e_idx = jnp.min(jnp.where(is_max, iota_e, E), axis=1, keepdims=True)
        oh = (iota_e == e_idx).astype(jnp.float32)

        r = lax.broadcasted_iota(jnp.int32, (TB, TB), 0)
        c = lax.broadcasted_iota(jnp.int32, (TB, TB), 1)
        tri = (c <= r).astype(jnp.float32)
        prefix = jnp.dot(tri, oh, preferred_element_type=jnp.float32) + carry_ref[...]
        carry_ref[...] = carry_ref[...] + jnp.sum(oh, axis=0, keepdims=True)

        pos = jnp.sum(prefix * oh, axis=1, keepdims=True) - 1.0
        valid = pos < C
        posi = jnp.minimum(pos, C - 1).astype(jnp.int32)
        r_iota = lax.broadcasted_iota(jnp.int32, (TB, 1), 0)
        slot = jnp.where(valid, e_idx * C + posi + C, r_iota & (C - 1))
        srow = slot.reshape(1, TB)
        grow = gate.reshape(1, TB)
        slot_ref[...] = srow
        gate_ref[...] = grow
        idx = i * TB
        slot_sc[0, pl.ds(idx, TB)] = srow[0, :]
        gate_sc[0, pl.ds(idx, TB)] = grow[0, :]

    @pl.when(i >= PH)
    def _():
        k = i - PH
        slot_d = slot_sc[...] - C
        s_iota = k * SB + lax.broadcasted_iota(jnp.int32, (SB, T), 0)
        mk = (s_iota == slot_d).astype(jnp.float32)
        t_iota = lax.broadcasted_iota(jnp.int32, (SB, T), 1).astype(jnp.float32)
        tos = jnp.sum(mk * t_iota, axis=1, keepdims=True).astype(jnp.int32)
        cnt = jnp.sum(mk, axis=1, keepdims=True)
        s_col = k * SB + lax.broadcasted_iota(jnp.int32, (SB, 1), 0)
        tos_ref[...] = jnp.where(cnt > 0.0, tos, s_col).reshape(1, SB)
        gs_ref[...] = jnp.sum(mk * gate_sc[...], axis=1, keepdims=True)


def _route_invert(x, w_router):
    return pl.pallas_call(
        _route_invert_body,
        grid=(PH + SLOTS // SB,),
        in_specs=[
            pl.BlockSpec((TB, D), lambda i: (jnp.minimum(i, PH - 1), 0)),
            pl.BlockSpec((D, E), lambda i: (0, 0)),
        ],
        out_specs=[
            pl.BlockSpec((1, TB), lambda i: (0, jnp.minimum(i, PH - 1))),
            pl.BlockSpec((1, TB), lambda i: (0, jnp.minimum(i, PH - 1))),
            pl.BlockSpec((1, SB), lambda i: (0, jnp.maximum(i - PH, 0))),
            pl.BlockSpec((SB, 1), lambda i: (jnp.maximum(i - PH, 0), 0)),
        ],
        out_shape=[
            jax.ShapeDtypeStruct((1, T), jnp.int32),
            jax.ShapeDtypeStruct((1, T), jnp.float32),
            jax.ShapeDtypeStruct((1, SLOTS), jnp.int32),
            jax.ShapeDtypeStruct((SLOTS, 1), jnp.float32),
        ],
        scratch_shapes=[
            pltpu.VMEM((1, E), jnp.float32),
            pltpu.VMEM((1, T), jnp.int32),
            pltpu.VMEM((1, T), jnp.float32),
        ],
    )(x, w_router)


def _dispatch_body(tos_hbm, x_hbm, ei_hbm, idx_v, rows_v, sem):
    cid = lax.axis_index("c")
    sid = lax.axis_index("s")
    base = (sid * NC + cid) * SPT
    pltpu.sync_copy(tos_hbm.at[pl.ds(base, SPT)], idx_v)
    pltpu.async_copy(x_hbm.at[idx_v], rows_v, sem).wait()
    pltpu.sync_copy(rows_v, ei_hbm.at[pl.ds(base, SPT)])


def _dispatch(tos, x):
    mesh = plsc.VectorSubcoreMesh(
        core_axis_name="c", subcore_axis_name="s", num_cores=NC, num_subcores=NS)
    return pl.kernel(
        _dispatch_body,
        out_type=jax.ShapeDtypeStruct((SLOTS, D), jnp.float32),
        mesh=mesh,
        compiler_params=pltpu.CompilerParams(needs_layout_passes=False),
        scratch_types=[
            pltpu.VMEM((SPT,), jnp.int32),
            pltpu.VMEM((SPT, D), jnp.float32),
            pltpu.SemaphoreType.DMA,
        ],
    )(tos, x)


def _ffn_body(ei_ref, w1_ref, b1_ref, w2_ref, b2_ref, gs_ref, eo_ref,
              obuf, sems):
    s = pl.program_id(0)
    buf = lax.rem(s, 2)

    @pl.when(s >= 2)
    def _():
        pltpu.make_async_copy(obuf.at[buf], eo_ref.at[s - 2], sems.at[buf]).wait()

    @pl.when(s == 0)
    def _():
        obuf[0] = jnp.zeros_like(obuf.at[0])

    @pl.when(s > 0)
    def _():
        a = ei_ref[0]
        h = jnp.dot(a, w1_ref[0], preferred_element_type=jnp.float32) + b1_ref[0]
        h = jnp.maximum(h, 0.0)
        o = jnp.dot(h, w2_ref[0], preferred_element_type=jnp.float32) + b2_ref[0]
        obuf[buf] = o * gs_ref[0]

    pltpu.make_async_copy(obuf.at[buf], eo_ref.at[s], sems.at[buf]).start()

    @pl.when(s == E)
    def _():
        pltpu.make_async_copy(obuf.at[buf], eo_ref.at[s], sems.at[buf]).wait()
        pltpu.make_async_copy(
            obuf.at[1 - buf], eo_ref.at[s - 1], sems.at[1 - buf]).wait()


def _ffn(ei, w1, b1, w2, b2, gs):
    em = lambda s: (jnp.maximum(s - 1, 0), 0, 0)
    return pl.pallas_call(
        _ffn_body,
        grid=(E + 1,),
        in_specs=[
            pl.BlockSpec((1, C, D), em),
            pl.BlockSpec((1, D, F), em),
            pl.BlockSpec((1, 1, F), em),
            pl.BlockSpec((1, F, D), em),
            pl.BlockSpec((1, 1, D), em),
            pl.BlockSpec((1, C, 1), em),
        ],
        out_specs=pl.BlockSpec(memory_space=pltpu.MemorySpace.HBM),
        out_shape=jax.ShapeDtypeStruct((E + 1, C, D), jnp.float32),
        scratch_shapes=[
            pltpu.VMEM((2, C, D), jnp.float32),
            pltpu.SemaphoreType.DMA((2,)),
        ],
    )(ei, w1, b1, w2, b2, gs)


def _combine_body(slot_hbm, eo_hbm, y_hbm, idx_v, rows_v, sem):
    cid = lax.axis_index("c")
    sid = lax.axis_index("s")
    base = (sid * NC + cid) * TPW
    pltpu.sync_copy(slot_hbm.at[pl.ds(base, TPW)], idx_v)
    pltpu.async_copy(eo_hbm.at[idx_v], rows_v, sem).wait()
    pltpu.sync_copy(rows_v, y_hbm.at[pl.ds(base, TPW)])


def _combine(slot, eo):
    mesh = plsc.VectorSubcoreMesh(
        core_axis_name="c", subcore_axis_name="s", num_cores=NC, num_subcores=NS)
    return pl.kernel(
        _combine_body,
        out_type=jax.ShapeDtypeStruct((T, D), jnp.float32),
        mesh=mesh,
        compiler_params=pltpu.CompilerParams(needs_layout_passes=False),
        scratch_types=[
            pltpu.VMEM((TPW,), jnp.int32),
            pltpu.VMEM((TPW, D), jnp.float32),
            pltpu.SemaphoreType.DMA,
        ],
    )(slot, eo)


def kernel(inputs, W_router, W1, b1, W2, b2):
    Bv, Sv, d = inputs.shape
    x = inputs.reshape(T, D)
    slot2, gate2, tos, gs = _route_invert(x, W_router)
    slot = slot2.reshape(T)
    ei = _dispatch(tos.reshape(SLOTS), x)
    eo = _ffn(ei.reshape(E, C, D), W1, b1.reshape(E, 1, F),
              W2, b2.reshape(E, 1, D), gs.reshape(E, C, 1))
    y = _combine(slot, eo.reshape((E + 1) * C, D))
    return y.reshape(Bv, Sv, d)


# speedup vs baseline: 1.2148x; 1.0015x over previous
import functools

import jax
import jax.numpy as jnp
from jax import lax
from jax.experimental import pallas as pl
from jax.experimental.pallas import tpu as pltpu
from jax.experimental.pallas import tpu_sc as plsc

E = 64
D = 768
F = 1024
T = 2048
C = max(int(round(1.0 * T / E)), 4)
SLOTS = E * C

TB = 256

NC = 2
NS = 16
L = 16
NW = NC * NS
SPT = SLOTS // NW
TPW = T // NW


SB = 256
PH = T // TB


def _route_invert_body(x_ref, wr_ref, slot_ref, gate_ref, tos_ref, gs_ref,
                       carry_ref, slot_sc, gate_sc):
    i = pl.program_id(0)

    @pl.when(i == 0)
    def _():
        carry_ref[...] = jnp.zeros_like(carry_ref)

    @pl.when(i < PH)
    def _():
        x = x_ref[...]
        logits = jnp.dot(x, wr_ref[...], preferred_element_type=jnp.float32)
        m = jnp.max(logits, axis=1, keepdims=True)
        s = jnp.sum(jnp.exp(logits - m), axis=1, keepdims=True)
        gate = 1.0 / s

        iota_e = lax.broadcasted_iota(jnp.int32, (TB, E), 1)
        is_max = logits == m
        e_idx = jnp.min(jnp.where(is_max, iota_e, E), axis=1, keepdims=True)
        oh = (iota_e == e_idx).astype(jnp.float32)

        r = lax.broadcasted_iota(jnp.int32, (TB, TB), 0)
        c = lax.broadcasted_iota(jnp.int32, (TB, TB), 1)
        tri = (c <= r).astype(jnp.float32)
        prefix = jnp.dot(tri, oh, preferred_element_type=jnp.float32) + carry_ref[...]
        carry_ref[...] = carry_ref[...] + jnp.sum(oh, axis=0, keepdims=True)

        pos = jnp.sum(prefix * oh, axis=1, keepdims=True) - 1.0
        valid = pos < C
        posi = jnp.minimum(pos, C - 1).astype(jnp.int32)
        r_iota = lax.broadcasted_iota(jnp.int32, (TB, 1), 0)
        slot = jnp.where(valid, e_idx * C + posi + C, r_iota & (C - 1))
        srow = slot.reshape(1, TB)
        grow = gate.reshape(1, TB)
        slot_ref[...] = srow
        gate_ref[...] = grow
        idx = i * TB
        slot_sc[0, pl.ds(idx, TB)] = srow[0, :]
        gate_sc[0, pl.ds(idx, TB)] = grow[0, :]

    @pl.when(i >= PH)
    def _():
        k = i - PH
        slot_d = slot_sc[...] - C
        s_iota = k * SB + lax.broadcasted_iota(jnp.int32, (SB, T), 0)
        mk = (s_iota == slot_d).astype(jnp.float32)
        t_iota = lax.broadcasted_iota(jnp.int32, (SB, T), 1).astype(jnp.float32)
        tos = jnp.sum(mk * t_iota, axis=1, keepdims=True).astype(jnp.int32)
        cnt = jnp.sum(mk, axis=1, keepdims=True)
        s_col = k * SB + lax.broadcasted_iota(jnp.int32, (SB, 1), 0)
        tos_ref[...] = jnp.where(cnt > 0.0, tos, s_col).reshape(1, SB)
        gs_ref[...] = jnp.sum(mk * gate_sc[...], axis=1, keepdims=True)


def _route_invert(x, w_router):
    return pl.pallas_call(
        _route_invert_body,
        grid=(PH + SLOTS // SB,),
        in_specs=[
            pl.BlockSpec((TB, D), lambda i: (jnp.minimum(i, PH - 1), 0)),
            pl.BlockSpec((D, E), lambda i: (0, 0)),
        ],
        out_specs=[
            pl.BlockSpec((1, TB), lambda i: (0, jnp.minimum(i, PH - 1))),
            pl.BlockSpec((1, TB), lambda i: (0, jnp.minimum(i, PH - 1))),
            pl.BlockSpec((1, SB), lambda i: (0, jnp.maximum(i - PH, 0))),
            pl.BlockSpec((SB, 1), lambda i: (jnp.maximum(i - PH, 0), 0)),
        ],
        out_shape=[
            jax.ShapeDtypeStruct((1, T), jnp.int32),
            jax.ShapeDtypeStruct((1, T), jnp.float32),
            jax.ShapeDtypeStruct((1, SLOTS), jnp.int32),
            jax.ShapeDtypeStruct((SLOTS, 1), jnp.float32),
        ],
        scratch_shapes=[
            pltpu.VMEM((1, E), jnp.float32),
            pltpu.VMEM((1, T), jnp.int32),
            pltpu.VMEM((1, T), jnp.float32),
        ],
    )(x, w_router)


def _dispatch_body(tos_hbm, x_hbm, ei_hbm, idx_v, rows_v, sem):
    cid = lax.axis_index("c")
    sid = lax.axis_index("s")
    base = (sid * NC + cid) * SPT
    pltpu.sync_copy(tos_hbm.at[pl.ds(base, SPT)], idx_v)
    pltpu.async_copy(x_hbm.at[idx_v], rows_v, sem).wait()
    pltpu.sync_copy(rows_v, ei_hbm.at[pl.ds(base, SPT)])


def _dispatch(tos, x):
    mesh = plsc.VectorSubcoreMesh(
        core_axis_name="c", subcore_axis_name="s", num_cores=NC, num_subcores=NS)
    return pl.kernel(
        _dispatch_body,
        out_type=jax.ShapeDtypeStruct((SLOTS, D), jnp.float32),
        mesh=mesh,
        compiler_params=pltpu.CompilerParams(needs_layout_passes=False),
        scratch_types=[
            pltpu.VMEM((SPT,), jnp.int32),
            pltpu.VMEM((SPT, D), jnp.float32),
            pltpu.SemaphoreType.DMA,
        ],
    )(tos, x)


def _ffn_body(ei_ref, w1_ref, b1_ref, w2_ref, b2_ref, gs_ref, eo_ref):
    s = pl.program_id(0)

    @pl.when(s == 0)
    def _():
        eo_ref[...] = jnp.zeros_like(eo_ref)

    @pl.when(s > 0)
    def _():
        a = ei_ref[0]
        h = jnp.dot(a, w1_ref[0], preferred_element_type=jnp.float32) + b1_ref[0]
        h = jnp.maximum(h, 0.0)
        o = jnp.dot(h, w2_ref[0], preferred_element_type=jnp.float32) + b2_ref[0]
        eo_ref[0] = o * gs_ref[0]


def _ffn(ei, w1, b1, w2, b2, gs):
    em = lambda s: (jnp.maximum(s - 1, 0), 0, 0)
    return pl.pallas_call(
        _ffn_body,
        grid=(E + 1,),
        in_specs=[
            pl.BlockSpec((1, C, D), em),
            pl.BlockSpec((1, D, F), em),
            pl.BlockSpec((1, 1, F), em),
            pl.BlockSpec((1, F, D), em),
            pl.BlockSpec((1, 1, D), em),
            pl.BlockSpec((1, C, 1), em),
        ],
        out_specs=pl.BlockSpec((1, C, D), lambda s: (s, 0, 0)),
        out_shape=jax.ShapeDtypeStruct((E + 1, C, D), jnp.float32),
    )(ei, w1, b1, w2, b2, gs)


def _combine_body(slot_hbm, eo_hbm, y_hbm, idx_v, rows_v, sem):
    cid = lax.axis_index("c")
    sid = lax.axis_index("s")
    base = (sid * NC + cid) * TPW
    pltpu.sync_copy(slot_hbm.at[pl.ds(base, TPW)], idx_v)
    pltpu.async_copy(eo_hbm.at[idx_v], rows_v, sem).wait()
    pltpu.sync_copy(rows_v, y_hbm.at[pl.ds(base, TPW)])


def _combine(slot, eo):
    mesh = plsc.VectorSubcoreMesh(
        core_axis_name="c", subcore_axis_name="s", num_cores=NC, num_subcores=NS)
    return pl.kernel(
        _combine_body,
        out_type=jax.ShapeDtypeStruct((T, D), jnp.float32),
        mesh=mesh,
        compiler_params=pltpu.CompilerParams(needs_layout_passes=False),
        scratch_types=[
            pltpu.VMEM((TPW,), jnp.int32),
            pltpu.VMEM((TPW, D), jnp.float32),
            pltpu.SemaphoreType.DMA,
        ],
    )(slot, eo)


def kernel(inputs, W_router, W1, b1, W2, b2):
    Bv, Sv, d = inputs.shape
    x = inputs.reshape(T, D)
    slot2, gate2, tos, gs = _route_invert(x, W_router)
    slot = slot2.reshape(T)
    ei = _dispatch(tos.reshape(SLOTS), x)
    eo = _ffn(ei.reshape(E, C, D), W1, b1.reshape(E, 1, F),
              W2, b2.reshape(E, 1, D), gs.reshape(E, C, 1))
    y = _combine(slot, eo.reshape((E + 1) * C, D))
    return y.reshape(Bv, Sv, d)
